# baseline (speedup 1.0000x reference)
import functools

import jax
import jax.numpy as jnp
from jax import lax
from jax.experimental import pallas as pl
from jax.experimental.pallas import tpu as pltpu
from jax.experimental.pallas import tpu_sc as plsc

B, S, L = 1024, 50, 8
T = B * S
EVENT_DIM, SKU_DIM, HIDDEN, ITEM_DIM, URL_DIM = 16, 64, 64, 64, 32
XW = 3 * 64 + 32

NUM_WORKERS = 32
SP = 128
WSUB = 56
SPAD = 56


NB = B // 2
TH = NB * S
TPW = TH // NUM_WORKERS
RPW = NB // NUM_WORKERS
SCHUNK = 50


def _sc_gather(sku_tbl, cat_tbl, url_tbl, word_tbl,
               sku_id, cat_id, url_id, word_id):
    mesh = plsc.VectorSubcoreMesh(core_axis_name="c", subcore_axis_name="s")

    @functools.partial(
        pl.kernel,
        out_type=jax.ShapeDtypeStruct((TH, XW), jnp.float32),
        mesh=mesh,
        compiler_params=pltpu.CompilerParams(use_tc_tiling_on_sc=False,
                                             needs_layout_passes=False),
        scratch_types=[
            pltpu.VMEM((RPW, SP), jnp.int32),
            pltpu.VMEM((RPW, SP), jnp.int32),
            pltpu.VMEM((RPW, SP), jnp.int32),
            pltpu.VMEM((WSUB, SP), jnp.int32),
            pltpu.VMEM((WSUB, SP), jnp.int32),
            pltpu.VMEM((S * L,), jnp.int32),
            pltpu.VMEM((S * L,), jnp.int32),
            pltpu.VMEM((S * L, 64), jnp.float32),
            pltpu.VMEM((S * L, 64), jnp.float32),
            pltpu.VMEM((SPAD, URL_DIM), jnp.float32),
            pltpu.VMEM((SPAD, URL_DIM), jnp.float32),
            pltpu.VMEM((S, 64), jnp.float32),
            pltpu.VMEM((S, 64), jnp.float32),
            pltpu.SemaphoreType.DMA,
            pltpu.SemaphoreType.DMA,
            pltpu.SemaphoreType.DMA,
            pltpu.SemaphoreType.DMA,
            pltpu.SemaphoreType.DMA,
            pltpu.SemaphoreType.DMA,
            pltpu.SemaphoreType.DMA,
        ],
    )
    def k(sku_tbl_h, cat_tbl_h, url_tbl_h, word_tbl_h,
          sku_id_h, cat_id_h, url_id_h, wid_h,
          x_out,
          skui, cati, urli, wb0, wb1, wx0, wx1,
          rb0, rb1, ub0, ub1, ac0, ac1,
          isem, gs0, gs1, ws0, ws1, is0, is1):
        wid = lax.axis_index("s") * 2 + lax.axis_index("c")
        base = wid * TPW
        brow = wid * RPW

        wb = (wb0, wb1)
        wx = (wx0, wx1)
        rb = (rb0, rb1)
        ub = (ub0, ub1)
        ac = (ac0, ac1)
        gsem = (gs0, gs1)
        wsem = (ws0, ws1)
        idsem = (is0, is1)

        d1 = pltpu.async_copy(sku_id_h.at[pl.ds(brow, RPW), :], skui, isem)
        d2 = pltpu.async_copy(cat_id_h.at[pl.ds(brow, RPW), :], cati, isem)
        d3 = pltpu.async_copy(url_id_h.at[pl.ds(brow, RPW), :], urli, isem)
        id_desc = [
            pltpu.async_copy(wid_h.at[brow + 0], wb0, idsem[0]),
            pltpu.async_copy(wid_h.at[brow + 1], wb1, idsem[1]),
        ]
        d1.wait(); d2.wait(); d3.wait()

        jobs = ([("sku", r) for r in range(RPW)]
                + [("cat", r) for r in range(RPW)]
                + [("url", r) for r in range(RPW)]
                + [("word", r) for r in range(RPW)])
        g_desc = [None, None]
        wr_desc = [None, None]
        lane = lax.iota(jnp.int32, 16)

        def start(k_):
            kind, r = jobs[k_]
            p = k_ % 2
            if wr_desc[p] is not None:
                wr_desc[p].wait()
                wr_desc[p] = None
            if kind == "sku":
                g_desc[p] = pltpu.async_copy(
                    sku_tbl_h.at[skui.at[r, pl.ds(0, SPAD)]],
                    rb[p].at[pl.ds(0, SPAD), :], gsem[p])
            elif kind == "cat":
                g_desc[p] = pltpu.async_copy(
                    cat_tbl_h.at[cati.at[r, pl.ds(0, SPAD)]],
                    rb[p].at[pl.ds(0, SPAD), :], gsem[p])
            elif kind == "url":
                g_desc[p] = pltpu.async_copy(
                    url_tbl_h.at[urli.at[r, pl.ds(0, SPAD)]], ub[p], gsem[p])
            else:
                id_desc[p].wait()
                wbuf = wb[p]
                wxp = wx[p]

                def compact(g, _):
                    j = lane + g * 16
                    v = plsc.load_gather(
                        wbuf, [lax.shift_right_logical(j, 3),
                               lax.bitwise_and(j, 7)])
                    wxp[pl.ds(g * 16, 16)] = v
                    return 0

                lax.fori_loop(0, S * L // 16, compact, 0, unroll=False)
                if r + 2 < RPW:
                    id_desc[p] = pltpu.async_copy(
                        wid_h.at[brow + r + 2], wbuf, idsem[p])
                g_desc[p] = pltpu.async_copy(
                    word_tbl_h.at[wxp], rb[p], gsem[p])

        def finish(k_):
            kind, r = jobs[k_]
            p = k_ % 2
            g_desc[p].wait()
            if kind == "sku":
                wr_desc[p] = pltpu.async_copy(
                    rb[p].at[pl.ds(0, S), :],
                    x_out.at[pl.ds(base + r * S, S), pl.ds(0, 64)], wsem[p])
            elif kind == "cat":
                wr_desc[p] = pltpu.async_copy(
                    rb[p].at[pl.ds(0, S), :],
                    x_out.at[pl.ds(base + r * S, S), pl.ds(64, 64)], wsem[p])
            elif kind == "url":
                wr_desc[p] = pltpu.async_copy(
                    ub[p].at[pl.ds(0, S), :],
                    x_out.at[pl.ds(base + r * S, S), pl.ds(192, URL_DIM)],
                    wsem[p])
            else:
                rbuf = rb[p]
                abuf = ac[p]

                def acc_body(t, _):
                    for d in range(ITEM_DIM // 16):
                        sl = pl.ds(d * 16, 16)
                        v = rbuf[t * L, sl]
                        for l in range(1, L):
                            v = v + rbuf[t * L + l, sl]
                        abuf[t, sl] = v * (1.0 / L)
                    return 0

                lax.fori_loop(0, S, acc_body, 0, unroll=False)
                wr_desc[p] = pltpu.async_copy(
                    abuf,
                    x_out.at[pl.ds(base + r * S, S), pl.ds(128, 64)],
                    wsem[p])

        start(0)
        start(1)
        for k_ in range(len(jobs)):
            finish(k_)
            if k_ + 2 < len(jobs):
                start(k_ + 2)
        for p in (0, 1):
            if wr_desc[p] is not None:
                wr_desc[p].wait()

    return k(sku_tbl, cat_tbl, url_tbl, word_tbl,
             sku_id, cat_id, url_id, word_id)


TBLOCK = 1024
EPS = 1e-5


def _ln_rows(x, g, b):
    mu = jnp.mean(x, axis=-1, keepdims=True)
    var = jnp.mean((x - mu) * (x - mu), axis=-1, keepdims=True)
    return (x - mu) * lax.rsqrt(var + EPS) * g + b


def _tc_body(ev_id_ref, pr_id_ref, x_ref,
             event_tbl_ref, price_tbl_ref,
             ev_gb_ref, seg_gb_ref, st2_gb_ref, price_gb_ref,
             segM_ref, segE_ref, st2M_ref, st2E_ref,
             sku_W_ref, fc1_W_ref, url_W_ref,
             out_ref):
    f32 = jnp.float32
    ev_id = ev_id_ref[...]
    pr_id = pr_id_ref[...]
    x = x_ref[...]

    mu = jnp.dot(x, segM_ref[...], preferred_element_type=f32)
    sq = jnp.dot(x * x, segM_ref[...], preferred_element_type=f32)
    r = lax.rsqrt(jnp.maximum(sq - mu * mu, 0.0) + EPS)
    mu_e = jnp.dot(mu, segE_ref[...], preferred_element_type=f32)
    r_e = jnp.dot(r, segE_ref[...], preferred_element_type=f32)
    xn = (x - mu_e) * r_e

    skun = xn[:, 0:64]
    catn = xn[:, 64:128]
    wordn = xn[:, 128:192]
    urln = xn[:, 192:224]

    word_g = seg_gb_ref[0:1, 128:192]
    word_b = seg_gb_ref[1:2, 128:192]
    word = wordn * word_g + word_b

    y_sku = jnp.dot(skun, sku_W_ref[0:64, :], preferred_element_type=f32) \
        + sku_W_ref[64:65, :]
    y_url = jnp.dot(urln, url_W_ref[0:URL_DIM, :], preferred_element_type=f32) \
        + url_W_ref[URL_DIM:URL_DIM + 1, :]
    y = jnp.concatenate([y_sku, y_url], axis=1)
    mu2 = jnp.dot(y, st2M_ref[...], preferred_element_type=f32)
    sq2 = jnp.dot(y * y, st2M_ref[...], preferred_element_type=f32)
    r2 = lax.rsqrt(jnp.maximum(sq2 - mu2 * mu2, 0.0) + EPS)
    mu2_e = jnp.dot(mu2, st2E_ref[...], preferred_element_type=f32)
    r2_e = jnp.dot(r2, st2E_ref[...], preferred_element_type=f32)
    yn = jnp.maximum((y - mu2_e) * r2_e * st2_gb_ref[0:1, :] + st2_gb_ref[1:2, :],
                     0.0)
    sku2 = yn[:, 0:64]
    url2 = yn[:, 64:128]

    ev_tbl = _ln_rows(event_tbl_ref[...], ev_gb_ref[0:1, :], ev_gb_ref[1:2, :])
    ev_oh = (lax.broadcasted_iota(jnp.int32, (TBLOCK, 8), 1) == ev_id)
    ev = jnp.dot(ev_oh.astype(f32), ev_tbl, preferred_element_type=f32)

    pr_tbl = _ln_rows(price_tbl_ref[...], price_gb_ref[0:1, :],
                      price_gb_ref[1:2, :])
    pr_fold = jnp.dot(pr_tbl, fc1_W_ref[128:192, :], preferred_element_type=f32)
    pr_oh = (lax.broadcasted_iota(jnp.int32, (TBLOCK, 128), 1) == pr_id)
    item = jnp.dot(pr_oh.astype(f32), pr_fold, preferred_element_type=f32)

    item = item + jnp.dot(sku2, fc1_W_ref[0:64, :], preferred_element_type=f32)
    item = item + jnp.dot(catn, fc1_W_ref[64:128, :], preferred_element_type=f32)
    item = item + jnp.dot(word, fc1_W_ref[192:256, :], preferred_element_type=f32)
    item = jnp.maximum(item + fc1_W_ref[256:257, :], 0.0)

    sku_m = (ev_id == 2) | (ev_id == 3) | (ev_id == 4)
    agg = (jnp.where(sku_m, item, 0.0)
           + jnp.where(ev_id == 5, url2, 0.0)
           + jnp.where(ev_id == 6, word, 0.0))
    out_ref[...] = jnp.concatenate([ev, agg], axis=1)


def _seg_mats(widths):
    tot = sum(widths)
    n = len(widths)
    M = jnp.zeros((tot, n), jnp.float32)
    E = jnp.zeros((n, tot), jnp.float32)
    off = 0
    for i, w in enumerate(widths):
        M = M.at[off:off + w, i].set(1.0 / w)
        E = E.at[i, off:off + w].set(1.0)
        off += w
    return M, E


def _tc_encode(p, ev_id, pr_id, x):
    grid = (TH // TBLOCK,)

    def tok2(d):
        return pl.BlockSpec((TBLOCK, d), lambda i: (i, 0))

    def whole(shape):
        return pl.BlockSpec(shape, lambda i: (0, 0))

    event_tbl = jnp.zeros((8, EVENT_DIM), jnp.float32).at[0:7].set(p['event_tbl'])
    price_tbl = jnp.zeros((128, HIDDEN), jnp.float32).at[0:100].set(p['price_tbl'])

    def pack_gb(g, b):
        return jnp.stack([g, b], axis=0)

    segM, segE = _seg_mats([64, 64, 64, 32])
    st2M, st2E = _seg_mats([64, 64])

    g1, b1 = p['sku_ln_g'], p['sku_ln_b']
    sku_W = jnp.concatenate(
        [g1[:, None] * p['sku_proj_W'],
         (p['sku_proj_b'] + b1 @ p['sku_proj_W'])[None, :]], axis=0)
    gu, bu = p['url_ln_g'], p['url_ln_b']
    url_W = jnp.concatenate(
        [gu[:, None] * p['url_proj_W'],
         (p['url_proj_b'] + bu @ p['url_proj_W'])[None, :]], axis=0)
    gc, bc = p['cat_ln_g'], p['cat_ln_b']
    W = p['fc1_W']
    fc1_W = jnp.concatenate(
        [W[0:64], gc[:, None] * W[64:128], W[128:192], W[192:256],
         (p['fc1_b'] + bc @ W[64:128])[None, :]], axis=0)

    st2_gb = jnp.concatenate(
        [pack_gb(p['sku_proj_ln_g'], p['sku_proj_ln_b']),
         pack_gb(p['url_proj_ln_g'], p['url_proj_ln_b'])], axis=1)
    seg_gb = jnp.concatenate(
        [pack_gb(g1, b1), pack_gb(gc, bc),
         pack_gb(p['word_ln_g'], p['word_ln_b']),
         pack_gb(gu, bu)], axis=1)

    args = (
        ev_id.reshape(TH, 1), pr_id.reshape(TH, 1), x,
        event_tbl, price_tbl,
        pack_gb(p['event_ln_g'], p['event_ln_b']),
        seg_gb, st2_gb,
        pack_gb(p['price_ln_g'], p['price_ln_b']),
        segM, segE, st2M, st2E,
        sku_W, fc1_W, url_W,
    )
    in_specs = [
        tok2(1), tok2(1), tok2(XW),
        whole((8, EVENT_DIM)), whole((128, HIDDEN)),
        whole((2, EVENT_DIM)), whole((2, XW)), whole((2, 128)),
        whole((2, HIDDEN)),
        whole((XW, 4)), whole((4, XW)), whole((128, 2)), whole((2, 128)),
        whole((65, HIDDEN)), whole((257, ITEM_DIM)), whole((33, ITEM_DIM)),
    ]
    return pl.pallas_call(
        _tc_body,
        grid=grid,
        in_specs=in_specs,
        out_specs=pl.BlockSpec((TBLOCK, EVENT_DIM + ITEM_DIM), lambda i: (i, 0)),
        out_shape=jax.ShapeDtypeStruct((TH, EVENT_DIM + ITEM_DIM), jnp.float32),
    )(*args)


def kernel(params, event_type, sku_id, url_id, cat_id, price_id, word_id):
    ev = event_type.astype(jnp.int32)
    pr = price_id.astype(jnp.int32)
    sku = jnp.pad(sku_id.astype(jnp.int32), ((0, 0), (0, SP - S)))
    cat = jnp.pad(cat_id.astype(jnp.int32), ((0, 0), (0, SP - S)))
    url = jnp.pad(url_id.astype(jnp.int32), ((0, 0), (0, SP - S)))
    wrd = jnp.pad(word_id.astype(jnp.int32),
                  ((0, 0), (0, WSUB - S), (0, SP - L)))

    halves = []
    for h in range(2):
        sl = slice(h * NB, (h + 1) * NB)
        x = _sc_gather(
            params['sku_tbl'], params['cat_tbl'], params['url_tbl'],
            params['word_tbl'],
            sku[sl], cat[sl], url[sl], wrd[sl])
        halves.append(
            _tc_encode(params, ev[sl].reshape(TH), pr[sl].reshape(TH), x))
    user_flat = jnp.concatenate(halves, axis=0)
    user_emb = user_flat.reshape(B, S, EVENT_DIM + ITEM_DIM)
    mask = event_type == 0
    return (user_emb, mask)


# speedup vs baseline: 1.0121x; 1.0121x over previous
import functools

import jax
import jax.numpy as jnp
from jax import lax
from jax.experimental import pallas as pl
from jax.experimental.pallas import tpu as pltpu
from jax.experimental.pallas import tpu_sc as plsc

B, S, L = 1024, 50, 8
T = B * S
EVENT_DIM, SKU_DIM, HIDDEN, ITEM_DIM, URL_DIM = 16, 64, 64, 64, 32
XW = 3 * 64 + 32

NUM_WORKERS = 32
SP = 128
WSUB = 56
SPAD = 56


NB = B
TH = NB * S
TPW = TH // NUM_WORKERS
RPW = NB // NUM_WORKERS
SCHUNK = 50


def _sc_compact_word(word_f32, rcidx):
    mesh = plsc.VectorSubcoreMesh(core_axis_name="c", subcore_axis_name="s")

    @functools.partial(
        pl.kernel,
        out_type=jax.ShapeDtypeStruct((B, S * L), jnp.int32),
        mesh=mesh,
        compiler_params=pltpu.CompilerParams(use_tc_tiling_on_sc=False,
                                             needs_layout_passes=False),
        scratch_types=[
            pltpu.VMEM((S, L), jnp.float32),
            pltpu.VMEM((S, L), jnp.float32),
            pltpu.VMEM((2, S * L // 16, 16), jnp.int32),
            pltpu.VMEM((S * L,), jnp.int32),
            pltpu.VMEM((S * L,), jnp.int32),
            pltpu.SemaphoreType.DMA,
            pltpu.SemaphoreType.DMA,
            pltpu.SemaphoreType.DMA,
        ],
    )
    def k(wid_h, rcidx_h, out, wb0, wb1, rcv, wx0, wx1, isem, s0, s1):
        w = lax.axis_index("s") * 2 + lax.axis_index("c")
        nrows = B // NUM_WORKERS
        brow = w * nrows
        wb = (wb0, wb1)
        wx = (wx0, wx1)
        osem = (s0, s1)

        pltpu.async_copy(rcidx_h, rcv, isem).wait()
        id_desc = [pltpu.async_copy(wid_h.at[brow + 0], wb0, isem),
                   pltpu.async_copy(wid_h.at[brow + 1], wb1, isem)]
        wr_desc = [None, None]
        for r in range(nrows):
            p_ = r % 2
            id_desc[p_].wait()
            wbuf = wb[p_]
            wxp = wx[p_]
            if wr_desc[p_] is not None:
                wr_desc[p_].wait()

            def compact(g, _):
                ri = rcv[0, g, :]
                ci = rcv[1, g, :]
                v = plsc.load_gather(wbuf, [ri, ci])
                wxp[pl.ds(g * 16, 16)] = plsc.bitcast(v, jnp.int32)
                return 0

            lax.fori_loop(0, S * L // 16, compact, 0, unroll=False)
            if r + 2 < nrows:
                id_desc[p_] = pltpu.async_copy(
                    wid_h.at[brow + r + 2], wbuf, isem)
            wr_desc[p_] = pltpu.async_copy(wxp, out.at[brow + r], osem[p_])
        for p_ in (0, 1):
            if wr_desc[p_] is not None:
                wr_desc[p_].wait()

    return k(word_f32, rcidx)


def _sc_gather(sku_tbl, cat_tbl, url_tbl, word_tbl,
               sku_id, cat_id, url_id, word_id):
    mesh = plsc.VectorSubcoreMesh(core_axis_name="c", subcore_axis_name="s")

    @functools.partial(
        pl.kernel,
        out_type=jax.ShapeDtypeStruct((TH, XW), jnp.float32),
        mesh=mesh,
        compiler_params=pltpu.CompilerParams(use_tc_tiling_on_sc=False),
        scratch_types=[
            pltpu.VMEM((RPW, SP), jnp.int32),
            pltpu.VMEM((RPW, SP), jnp.int32),
            pltpu.VMEM((RPW, SP), jnp.int32),
            pltpu.VMEM((S * L,), jnp.int32),
            pltpu.VMEM((S * L,), jnp.int32),
            pltpu.VMEM((S * L, 64), jnp.float32),
            pltpu.VMEM((S * L, 64), jnp.float32),
            pltpu.VMEM((SPAD, URL_DIM), jnp.float32),
            pltpu.VMEM((SPAD, URL_DIM), jnp.float32),
            pltpu.VMEM((S, 64), jnp.float32),
            pltpu.VMEM((S, 64), jnp.float32),
            pltpu.SemaphoreType.DMA,
            pltpu.SemaphoreType.DMA,
            pltpu.SemaphoreType.DMA,
            pltpu.SemaphoreType.DMA,
            pltpu.SemaphoreType.DMA,
            pltpu.SemaphoreType.DMA,
            pltpu.SemaphoreType.DMA,
        ],
    )
    def k(sku_tbl_h, cat_tbl_h, url_tbl_h, word_tbl_h,
          sku_id_h, cat_id_h, url_id_h, wid_h,
          x_out,
          skui, cati, urli, wx0, wx1,
          rb0, rb1, ub0, ub1, ac0, ac1,
          isem, gs0, gs1, ws0, ws1, is0, is1):
        wid = lax.axis_index("s") * 2 + lax.axis_index("c")
        base = wid * TPW
        brow = wid * RPW

        rb = (rb0, rb1)
        wx = (wx0, wx1)
        ub = (ub0, ub1)
        ac = (ac0, ac1)
        gsem = (gs0, gs1)
        wsem = (ws0, ws1)
        idsem = (is0, is1)

        d1 = pltpu.async_copy(sku_id_h.at[pl.ds(brow, RPW), :], skui, isem)
        d2 = pltpu.async_copy(cat_id_h.at[pl.ds(brow, RPW), :], cati, isem)
        d3 = pltpu.async_copy(url_id_h.at[pl.ds(brow, RPW), :], urli, isem)
        id_desc = [None, None]
        d1.wait(); d2.wait(); d3.wait()

        jobs = ([("sku", r) for r in range(RPW)]
                + [("cat", r) for r in range(RPW)]
                + [("url", r) for r in range(RPW)]
                + [("word", r) for r in range(RPW)])
        g_desc = [None, None]
        wr_desc = [None, None]

        def start(k_):
            kind, r = jobs[k_]
            p = k_ % 2
            if wr_desc[p] is not None:
                wr_desc[p].wait()
                wr_desc[p] = None
            if kind == "sku":
                g_desc[p] = pltpu.async_copy(
                    sku_tbl_h.at[skui.at[r, pl.ds(0, SPAD)]],
                    rb[p].at[pl.ds(0, SPAD), :], gsem[p])
            elif kind == "cat":
                g_desc[p] = pltpu.async_copy(
                    cat_tbl_h.at[cati.at[r, pl.ds(0, SPAD)]],
                    rb[p].at[pl.ds(0, SPAD), :], gsem[p])
            elif kind == "url":
                g_desc[p] = pltpu.async_copy(
                    url_tbl_h.at[urli.at[r, pl.ds(0, SPAD)]], ub[p], gsem[p])
            else:
                if id_desc[p] is None:
                    id_desc[p] = pltpu.async_copy(
                        wid_h.at[brow + r], wx[p], idsem[p])
                id_desc[p].wait()
                g_desc[p] = pltpu.async_copy(
                    word_tbl_h.at[wx[p]], rb[p], gsem[p])

        def finish(k_):
            kind, r = jobs[k_]
            p = k_ % 2
            g_desc[p].wait()
            if kind == "sku":
                wr_desc[p] = pltpu.async_copy(
                    rb[p].at[pl.ds(0, S), :],
                    x_out.at[pl.ds(base + r * S, S), pl.ds(0, 64)], wsem[p])
            elif kind == "cat":
                wr_desc[p] = pltpu.async_copy(
                    rb[p].at[pl.ds(0, S), :],
                    x_out.at[pl.ds(base + r * S, S), pl.ds(64, 64)], wsem[p])
            elif kind == "url":
                wr_desc[p] = pltpu.async_copy(
                    ub[p].at[pl.ds(0, S), :],
                    x_out.at[pl.ds(base + r * S, S), pl.ds(192, URL_DIM)],
                    wsem[p])
            else:
                if r + 2 < RPW:
                    id_desc[p] = pltpu.async_copy(
                        wid_h.at[brow + r + 2], wx[p], idsem[p])
                else:
                    id_desc[p] = None
                rbuf = rb[p]
                abuf = ac[p]

                def acc_body(t, _):
                    for d in range(ITEM_DIM // 16):
                        sl = pl.ds(d * 16, 16)
                        v = rbuf[t * L, sl]
                        for l in range(1, L):
                            v = v + rbuf[t * L + l, sl]
                        abuf[t, sl] = v * (1.0 / L)
                    return 0

                lax.fori_loop(0, S, acc_body, 0, unroll=False)
                wr_desc[p] = pltpu.async_copy(
                    abuf,
                    x_out.at[pl.ds(base + r * S, S), pl.ds(128, 64)],
                    wsem[p])

        start(0)
        start(1)
        for k_ in range(len(jobs)):
            finish(k_)
            if k_ + 2 < len(jobs):
                start(k_ + 2)
        for p in (0, 1):
            if wr_desc[p] is not None:
                wr_desc[p].wait()

    return k(sku_tbl, cat_tbl, url_tbl, word_tbl,
             sku_id, cat_id, url_id, word_id)


TBLOCK = 1024
EPS = 1e-5


def _ln_rows(x, g, b):
    mu = jnp.mean(x, axis=-1, keepdims=True)
    var = jnp.mean((x - mu) * (x - mu), axis=-1, keepdims=True)
    return (x - mu) * lax.rsqrt(var + EPS) * g + b


def _tc_body(ev_id_ref, pr_id_ref, x_ref,
             event_tbl_ref, price_tbl_ref,
             ev_gb_ref, seg_gb_ref, st2_gb_ref, price_gb_ref,
             segM_ref, segE_ref, st2M_ref, st2E_ref,
             sku_W_ref, fc1_W_ref, url_W_ref,
             out_ref):
    f32 = jnp.float32
    ev_id = ev_id_ref[...]
    pr_id = pr_id_ref[...]
    x = x_ref[...]

    mu = jnp.dot(x, segM_ref[...], preferred_element_type=f32)
    sq = jnp.dot(x * x, segM_ref[...], preferred_element_type=f32)
    r = lax.rsqrt(jnp.maximum(sq - mu * mu, 0.0) + EPS)
    mu_e = jnp.dot(mu, segE_ref[...], preferred_element_type=f32)
    r_e = jnp.dot(r, segE_ref[...], preferred_element_type=f32)
    xn = (x - mu_e) * r_e

    skun = xn[:, 0:64]
    catn = xn[:, 64:128]
    wordn = xn[:, 128:192]
    urln = xn[:, 192:224]

    word_g = seg_gb_ref[0:1, 128:192]
    word_b = seg_gb_ref[1:2, 128:192]
    word = wordn * word_g + word_b

    y_sku = jnp.dot(skun, sku_W_ref[0:64, :], preferred_element_type=f32) \
        + sku_W_ref[64:65, :]
    y_url = jnp.dot(urln, url_W_ref[0:URL_DIM, :], preferred_element_type=f32) \
        + url_W_ref[URL_DIM:URL_DIM + 1, :]
    y = jnp.concatenate([y_sku, y_url], axis=1)
    mu2 = jnp.dot(y, st2M_ref[...], preferred_element_type=f32)
    sq2 = jnp.dot(y * y, st2M_ref[...], preferred_element_type=f32)
    r2 = lax.rsqrt(jnp.maximum(sq2 - mu2 * mu2, 0.0) + EPS)
    mu2_e = jnp.dot(mu2, st2E_ref[...], preferred_element_type=f32)
    r2_e = jnp.dot(r2, st2E_ref[...], preferred_element_type=f32)
    yn = jnp.maximum((y - mu2_e) * r2_e * st2_gb_ref[0:1, :] + st2_gb_ref[1:2, :],
                     0.0)
    sku2 = yn[:, 0:64]
    url2 = yn[:, 64:128]

    ev_tbl = _ln_rows(event_tbl_ref[...], ev_gb_ref[0:1, :], ev_gb_ref[1:2, :])
    ev_oh = (lax.broadcasted_iota(jnp.int32, (TBLOCK, 8), 1) == ev_id)
    ev = jnp.dot(ev_oh.astype(f32), ev_tbl, preferred_element_type=f32)

    pr_tbl = _ln_rows(price_tbl_ref[...], price_gb_ref[0:1, :],
                      price_gb_ref[1:2, :])
    pr_fold = jnp.dot(pr_tbl, fc1_W_ref[128:192, :], preferred_element_type=f32)
    pr_oh = (lax.broadcasted_iota(jnp.int32, (TBLOCK, 128), 1) == pr_id)
    item = jnp.dot(pr_oh.astype(f32), pr_fold, preferred_element_type=f32)

    item = item + jnp.dot(sku2, fc1_W_ref[0:64, :], preferred_element_type=f32)
    item = item + jnp.dot(catn, fc1_W_ref[64:128, :], preferred_element_type=f32)
    item = item + jnp.dot(word, fc1_W_ref[192:256, :], preferred_element_type=f32)
    item = jnp.maximum(item + fc1_W_ref[256:257, :], 0.0)

    sku_m = (ev_id == 2) | (ev_id == 3) | (ev_id == 4)
    agg = (jnp.where(sku_m, item, 0.0)
           + jnp.where(ev_id == 5, url2, 0.0)
           + jnp.where(ev_id == 6, word, 0.0))
    out_ref[...] = jnp.concatenate([ev, agg], axis=1)


def _seg_mats(widths):
    tot = sum(widths)
    n = len(widths)
    M = jnp.zeros((tot, n), jnp.float32)
    E = jnp.zeros((n, tot), jnp.float32)
    off = 0
    for i, w in enumerate(widths):
        M = M.at[off:off + w, i].set(1.0 / w)
        E = E.at[i, off:off + w].set(1.0)
        off += w
    return M, E


def _tc_encode(p, ev_id, pr_id, x):
    grid = (TH // TBLOCK,)

    def tok2(d):
        return pl.BlockSpec((TBLOCK, d), lambda i: (i, 0))

    def whole(shape):
        return pl.BlockSpec(shape, lambda i: (0, 0))

    event_tbl = jnp.zeros((8, EVENT_DIM), jnp.float32).at[0:7].set(p['event_tbl'])
    price_tbl = jnp.zeros((128, HIDDEN), jnp.float32).at[0:100].set(p['price_tbl'])

    def pack_gb(g, b):
        return jnp.stack([g, b], axis=0)

    segM, segE = _seg_mats([64, 64, 64, 32])
    st2M, st2E = _seg_mats([64, 64])

    g1, b1 = p['sku_ln_g'], p['sku_ln_b']
    sku_W = jnp.concatenate(
        [g1[:, None] * p['sku_proj_W'],
         (p['sku_proj_b'] + b1 @ p['sku_proj_W'])[None, :]], axis=0)
    gu, bu = p['url_ln_g'], p['url_ln_b']
    url_W = jnp.concatenate(
        [gu[:, None] * p['url_proj_W'],
         (p['url_proj_b'] + bu @ p['url_proj_W'])[None, :]], axis=0)
    gc, bc = p['cat_ln_g'], p['cat_ln_b']
    W = p['fc1_W']
    fc1_W = jnp.concatenate(
        [W[0:64], gc[:, None] * W[64:128], W[128:192], W[192:256],
         (p['fc1_b'] + bc @ W[64:128])[None, :]], axis=0)

    st2_gb = jnp.concatenate(
        [pack_gb(p['sku_proj_ln_g'], p['sku_proj_ln_b']),
         pack_gb(p['url_proj_ln_g'], p['url_proj_ln_b'])], axis=1)
    seg_gb = jnp.concatenate(
        [pack_gb(g1, b1), pack_gb(gc, bc),
         pack_gb(p['word_ln_g'], p['word_ln_b']),
         pack_gb(gu, bu)], axis=1)

    args = (
        ev_id.reshape(TH, 1), pr_id.reshape(TH, 1), x,
        event_tbl, price_tbl,
        pack_gb(p['event_ln_g'], p['event_ln_b']),
        seg_gb, st2_gb,
        pack_gb(p['price_ln_g'], p['price_ln_b']),
        segM, segE, st2M, st2E,
        sku_W, fc1_W, url_W,
    )
    in_specs = [
        tok2(1), tok2(1), tok2(XW),
        whole((8, EVENT_DIM)), whole((128, HIDDEN)),
        whole((2, EVENT_DIM)), whole((2, XW)), whole((2, 128)),
        whole((2, HIDDEN)),
        whole((XW, 4)), whole((4, XW)), whole((128, 2)), whole((2, 128)),
        whole((65, HIDDEN)), whole((257, ITEM_DIM)), whole((33, ITEM_DIM)),
    ]
    return pl.pallas_call(
        _tc_body,
        grid=grid,
        in_specs=in_specs,
        out_specs=pl.BlockSpec((TBLOCK, EVENT_DIM + ITEM_DIM), lambda i: (i, 0)),
        out_shape=jax.ShapeDtypeStruct((TH, EVENT_DIM + ITEM_DIM), jnp.float32),
    )(*args)


def kernel(params, event_type, sku_id, url_id, cat_id, price_id, word_id):
    ev = event_type.astype(jnp.int32)
    pr = price_id.astype(jnp.int32)
    sku = jnp.pad(sku_id.astype(jnp.int32), ((0, 0), (0, SP - S)))
    cat = jnp.pad(cat_id.astype(jnp.int32), ((0, 0), (0, SP - S)))
    url = jnp.pad(url_id.astype(jnp.int32), ((0, 0), (0, SP - S)))
    wrd_f = jax.lax.bitcast_convert_type(word_id.astype(jnp.int32), jnp.float32)

    j = jnp.arange(S * L, dtype=jnp.int32).reshape(S * L // 16, 16)
    rcidx = jnp.stack([j // L, j % L], axis=0)
    wrd = _sc_compact_word(wrd_f, rcidx)

    x = _sc_gather(
        params['sku_tbl'], params['cat_tbl'], params['url_tbl'],
        params['word_tbl'], sku, cat, url, wrd)
    user_flat = _tc_encode(params, ev.reshape(TH), pr.reshape(TH), x)
    user_emb = user_flat.reshape(B, S, EVENT_DIM + ITEM_DIM)
    mask = event_type == 0
    return (user_emb, mask)


# speedup vs baseline: 1.3333x; 1.3174x over previous
import functools

import jax
import jax.numpy as jnp
from jax import lax
from jax.experimental import pallas as pl
from jax.experimental.pallas import tpu as pltpu
from jax.experimental.pallas import tpu_sc as plsc

B, S, L = 1024, 50, 8
T = B * S
EVENT_DIM, SKU_DIM, HIDDEN, ITEM_DIM, URL_DIM = 16, 64, 64, 64, 32
XW = 3 * 64 + 32

NUM_WORKERS = 32
TPW = T // NUM_WORKERS


GCH = 400
WCH = 64


def _sc_gather(sku_tbl, cat_tbl, url_tbl, word_tbl,
               sku_id, cat_id, url_id, word_id_flat):
    mesh = plsc.VectorSubcoreMesh(core_axis_name="c", subcore_axis_name="s")

    @functools.partial(
        pl.kernel,
        out_type=jax.ShapeDtypeStruct((T, XW), jnp.float32),
        mesh=mesh,
        compiler_params=pltpu.CompilerParams(use_tc_tiling_on_sc=False),
        scratch_types=[
            pltpu.VMEM((TPW,), jnp.int32),
            pltpu.VMEM((TPW,), jnp.int32),
            pltpu.VMEM((TPW,), jnp.int32),
            pltpu.VMEM((TPW * L,), jnp.int32),
            pltpu.VMEM((WCH * L, 64), jnp.float32),
            pltpu.VMEM((WCH * L, 64), jnp.float32),
            pltpu.VMEM((GCH, URL_DIM), jnp.float32),
            pltpu.VMEM((GCH, URL_DIM), jnp.float32),
            pltpu.VMEM((WCH, 64), jnp.float32),
            pltpu.VMEM((WCH, 64), jnp.float32),
            pltpu.SemaphoreType.DMA,
            pltpu.SemaphoreType.DMA,
            pltpu.SemaphoreType.DMA,
            pltpu.SemaphoreType.DMA,
            pltpu.SemaphoreType.DMA,
        ],
    )
    def k(sku_tbl_h, cat_tbl_h, url_tbl_h, word_tbl_h,
          sku_id_h, cat_id_h, url_id_h, wid_h,
          x_out,
          skui, cati, urli, wordi, rb0, rb1, ub0, ub1, ac0, ac1,
          isem, gs0, gs1, ws0, ws1):
        wid = lax.axis_index("s") * 2 + lax.axis_index("c")
        base = wid * TPW

        rb = (rb0, rb1)
        ub = (ub0, ub1)
        ac = (ac0, ac1)
        gsem = (gs0, gs1)
        wsem = (ws0, ws1)

        d1 = pltpu.async_copy(sku_id_h.at[pl.ds(base, TPW)], skui, isem)
        d2 = pltpu.async_copy(cat_id_h.at[pl.ds(base, TPW)], cati, isem)
        d3 = pltpu.async_copy(url_id_h.at[pl.ds(base, TPW)], urli, isem)
        d4 = pltpu.async_copy(wid_h.at[pl.ds(base * L, TPW * L)], wordi, isem)
        d1.wait(); d2.wait(); d3.wait(); d4.wait()

        jobs = ([("sku", j) for j in range(TPW // GCH)]
                + [("cat", j) for j in range(TPW // GCH)]
                + [("url", j) for j in range(TPW // GCH)]
                + [("word", c) for c in range(TPW // WCH)])
        g_desc = [None, None]
        wr_desc = [None, None]

        def start(k_):
            kind, j = jobs[k_]
            p = k_ % 2
            if wr_desc[p] is not None:
                wr_desc[p].wait()
                wr_desc[p] = None
            if kind == "sku":
                g_desc[p] = pltpu.async_copy(
                    sku_tbl_h.at[skui.at[pl.ds(j * GCH, GCH)]],
                    rb[p].at[pl.ds(0, GCH), :], gsem[p])
            elif kind == "cat":
                g_desc[p] = pltpu.async_copy(
                    cat_tbl_h.at[cati.at[pl.ds(j * GCH, GCH)]],
                    rb[p].at[pl.ds(0, GCH), :], gsem[p])
            elif kind == "url":
                g_desc[p] = pltpu.async_copy(
                    url_tbl_h.at[urli.at[pl.ds(j * GCH, GCH)]], ub[p], gsem[p])
            else:
                g_desc[p] = pltpu.async_copy(
                    word_tbl_h.at[wordi.at[pl.ds(j * WCH * L, WCH * L)]],
                    rb[p], gsem[p])

        def finish(k_):
            kind, j = jobs[k_]
            p = k_ % 2
            g_desc[p].wait()
            if kind == "sku":
                wr_desc[p] = pltpu.async_copy(
                    rb[p].at[pl.ds(0, GCH), :],
                    x_out.at[pl.ds(base + j * GCH, GCH), pl.ds(0, 64)], wsem[p])
            elif kind == "cat":
                wr_desc[p] = pltpu.async_copy(
                    rb[p].at[pl.ds(0, GCH), :],
                    x_out.at[pl.ds(base + j * GCH, GCH), pl.ds(64, 64)], wsem[p])
            elif kind == "url":
                wr_desc[p] = pltpu.async_copy(
                    ub[p],
                    x_out.at[pl.ds(base + j * GCH, GCH), pl.ds(192, URL_DIM)],
                    wsem[p])
            else:
                rbuf = rb[p]
                abuf = ac[p]

                def acc_body(t, _):
                    for d in range(ITEM_DIM // 16):
                        sl = pl.ds(d * 16, 16)
                        v = rbuf[t * L, sl]
                        for l in range(1, L):
                            v = v + rbuf[t * L + l, sl]
                        abuf[t, sl] = v * (1.0 / L)
                    return 0

                lax.fori_loop(0, WCH, acc_body, 0, unroll=False)
                wr_desc[p] = pltpu.async_copy(
                    abuf,
                    x_out.at[pl.ds(base + j * WCH, WCH), pl.ds(128, 64)],
                    wsem[p])

        start(0)
        start(1)
        for k_ in range(len(jobs)):
            finish(k_)
            if k_ + 2 < len(jobs):
                start(k_ + 2)
        for p in (0, 1):
            if wr_desc[p] is not None:
                wr_desc[p].wait()

    return k(sku_tbl, cat_tbl, url_tbl, word_tbl,
             sku_id, cat_id, url_id, word_id_flat)


TBLOCK = 1024
EPS = 1e-5


def _ln_rows(x, g, b):
    mu = jnp.mean(x, axis=-1, keepdims=True)
    var = jnp.mean((x - mu) * (x - mu), axis=-1, keepdims=True)
    return (x - mu) * lax.rsqrt(var + EPS) * g + b


def _tc_body(ev_id_ref, pr_id_ref, x_ref,
             event_tbl_ref, price_tbl_ref,
             ev_gb_ref, seg_gb_ref, st2_gb_ref, price_gb_ref,
             segM_ref, segE_ref, st2M_ref, st2E_ref,
             sku_W_ref, fc1_W_ref, url_W_ref,
             out_ref):
    f32 = jnp.float32
    ev_id = ev_id_ref[...]
    pr_id = pr_id_ref[...]
    x = x_ref[...]

    mu = jnp.dot(x, segM_ref[...], preferred_element_type=f32)
    sq = jnp.dot(x * x, segM_ref[...], preferred_element_type=f32)
    r = lax.rsqrt(jnp.maximum(sq - mu * mu, 0.0) + EPS)
    mu_e = jnp.dot(mu, segE_ref[...], preferred_element_type=f32)
    r_e = jnp.dot(r, segE_ref[...], preferred_element_type=f32)
    xn = (x - mu_e) * r_e

    skun = xn[:, 0:64]
    catn = xn[:, 64:128]
    wordn = xn[:, 128:192]
    urln = xn[:, 192:224]

    word_g = seg_gb_ref[0:1, 128:192]
    word_b = seg_gb_ref[1:2, 128:192]
    word = wordn * word_g + word_b

    y_sku = jnp.dot(skun, sku_W_ref[0:64, :], preferred_element_type=f32) \
        + sku_W_ref[64:65, :]
    y_url = jnp.dot(urln, url_W_ref[0:URL_DIM, :], preferred_element_type=f32) \
        + url_W_ref[URL_DIM:URL_DIM + 1, :]
    y = jnp.concatenate([y_sku, y_url], axis=1)
    mu2 = jnp.dot(y, st2M_ref[...], preferred_element_type=f32)
    sq2 = jnp.dot(y * y, st2M_ref[...], preferred_element_type=f32)
    r2 = lax.rsqrt(jnp.maximum(sq2 - mu2 * mu2, 0.0) + EPS)
    mu2_e = jnp.dot(mu2, st2E_ref[...], preferred_element_type=f32)
    r2_e = jnp.dot(r2, st2E_ref[...], preferred_element_type=f32)
    yn = jnp.maximum((y - mu2_e) * r2_e * st2_gb_ref[0:1, :] + st2_gb_ref[1:2, :],
                     0.0)
    sku2 = yn[:, 0:64]
    url2 = yn[:, 64:128]

    ev_tbl = _ln_rows(event_tbl_ref[...], ev_gb_ref[0:1, :], ev_gb_ref[1:2, :])
    ev_oh = (lax.broadcasted_iota(jnp.int32, (TBLOCK, 8), 1) == ev_id)
    ev = jnp.dot(ev_oh.astype(f32), ev_tbl, preferred_element_type=f32)

    pr_tbl = _ln_rows(price_tbl_ref[...], price_gb_ref[0:1, :],
                      price_gb_ref[1:2, :])
    pr_fold = jnp.dot(pr_tbl, fc1_W_ref[128:192, :], preferred_element_type=f32)
    pr_oh = (lax.broadcasted_iota(jnp.int32, (TBLOCK, 128), 1) == pr_id)
    item = jnp.dot(pr_oh.astype(f32), pr_fold, preferred_element_type=f32)

    item = item + jnp.dot(sku2, fc1_W_ref[0:64, :], preferred_element_type=f32)
    item = item + jnp.dot(catn, fc1_W_ref[64:128, :], preferred_element_type=f32)
    item = item + jnp.dot(word, fc1_W_ref[192:256, :], preferred_element_type=f32)
    item = jnp.maximum(item + fc1_W_ref[256:257, :], 0.0)

    sku_m = (ev_id == 2) | (ev_id == 3) | (ev_id == 4)
    agg = (jnp.where(sku_m, item, 0.0)
           + jnp.where(ev_id == 5, url2, 0.0)
           + jnp.where(ev_id == 6, word, 0.0))
    out_ref[...] = jnp.concatenate([ev, agg], axis=1)


def _seg_mats(widths):
    tot = sum(widths)
    n = len(widths)
    M = jnp.zeros((tot, n), jnp.float32)
    E = jnp.zeros((n, tot), jnp.float32)
    off = 0
    for i, w in enumerate(widths):
        M = M.at[off:off + w, i].set(1.0 / w)
        E = E.at[i, off:off + w].set(1.0)
        off += w
    return M, E


def _tc_encode(p, ev_id, pr_id, x):
    grid = (T // TBLOCK,)

    def tok2(d):
        return pl.BlockSpec((TBLOCK, d), lambda i: (i, 0))

    def whole(shape):
        return pl.BlockSpec(shape, lambda i: (0, 0))

    event_tbl = jnp.zeros((8, EVENT_DIM), jnp.float32).at[0:7].set(p['event_tbl'])
    price_tbl = jnp.zeros((128, HIDDEN), jnp.float32).at[0:100].set(p['price_tbl'])

    def pack_gb(g, b):
        return jnp.stack([g, b], axis=0)

    segM, segE = _seg_mats([64, 64, 64, 32])
    st2M, st2E = _seg_mats([64, 64])

    g1, b1 = p['sku_ln_g'], p['sku_ln_b']
    sku_W = jnp.concatenate(
        [g1[:, None] * p['sku_proj_W'],
         (p['sku_proj_b'] + b1 @ p['sku_proj_W'])[None, :]], axis=0)
    gu, bu = p['url_ln_g'], p['url_ln_b']
    url_W = jnp.concatenate(
        [gu[:, None] * p['url_proj_W'],
         (p['url_proj_b'] + bu @ p['url_proj_W'])[None, :]], axis=0)
    gc, bc = p['cat_ln_g'], p['cat_ln_b']
    W = p['fc1_W']
    fc1_W = jnp.concatenate(
        [W[0:64], gc[:, None] * W[64:128], W[128:192], W[192:256],
         (p['fc1_b'] + bc @ W[64:128])[None, :]], axis=0)

    st2_gb = jnp.concatenate(
        [pack_gb(p['sku_proj_ln_g'], p['sku_proj_ln_b']),
         pack_gb(p['url_proj_ln_g'], p['url_proj_ln_b'])], axis=1)
    seg_gb = jnp.concatenate(
        [pack_gb(g1, b1), pack_gb(gc, bc),
         pack_gb(p['word_ln_g'], p['word_ln_b']),
         pack_gb(gu, bu)], axis=1)

    args = (
        ev_id.reshape(T, 1), pr_id.reshape(T, 1), x,
        event_tbl, price_tbl,
        pack_gb(p['event_ln_g'], p['event_ln_b']),
        seg_gb, st2_gb,
        pack_gb(p['price_ln_g'], p['price_ln_b']),
        segM, segE, st2M, st2E,
        sku_W, fc1_W, url_W,
    )
    in_specs = [
        tok2(1), tok2(1), tok2(XW),
        whole((8, EVENT_DIM)), whole((128, HIDDEN)),
        whole((2, EVENT_DIM)), whole((2, XW)), whole((2, 128)),
        whole((2, HIDDEN)),
        whole((XW, 4)), whole((4, XW)), whole((128, 2)), whole((2, 128)),
        whole((65, HIDDEN)), whole((257, ITEM_DIM)), whole((33, ITEM_DIM)),
    ]
    return pl.pallas_call(
        _tc_body,
        grid=grid,
        in_specs=in_specs,
        out_specs=pl.BlockSpec((TBLOCK, EVENT_DIM + ITEM_DIM), lambda i: (i, 0)),
        out_shape=jax.ShapeDtypeStruct((T, EVENT_DIM + ITEM_DIM), jnp.float32),
    )(*args)


def kernel(params, event_type, sku_id, url_id, cat_id, price_id, word_id):
    ev = event_type.astype(jnp.int32)
    x = _sc_gather(
        params['sku_tbl'], params['cat_tbl'], params['url_tbl'],
        params['word_tbl'],
        sku_id.astype(jnp.int32).reshape(T),
        cat_id.astype(jnp.int32).reshape(T),
        url_id.astype(jnp.int32).reshape(T),
        word_id.astype(jnp.int32).reshape(T * L),
    )
    user_flat = _tc_encode(params, ev.reshape(T),
                           price_id.astype(jnp.int32).reshape(T), x)
    user_emb = user_flat.reshape(B, S, EVENT_DIM + ITEM_DIM)
    mask = event_type == 0
    return (user_emb, mask)
